# baseline (speedup 1.0000x reference)
import jax
import jax.numpy as jnp
from jax import lax
from jax.experimental import pallas as pl
from jax.experimental.pallas import tpu as pltpu
from jax.experimental.pallas import tpu_sc as plsc

N = 10000
E = 320000
IN_C = 128
OUT_C = 128
DIM = 2
KS = 5
K = KS ** DIM
S = 2 ** DIM
KP = KS * (KS - 1)

NW = 32
EPW = E // NW
CHUNK = 32
NCH = 314
EPW_PAD = NCH * CHUNK
NPAD = 10240
ROWS_PER_TILE = NPAD // 16
LANES = 16


def _basis_body(attr_ref, src_ref, gidx_ref, bw_ref):
    a0 = attr_ref[0:1, :]
    a1 = attr_ref[1:2, :]
    src = src_ref[0:1, :]
    v0 = a0 * (KS - 1.0)
    v1 = a1 * (KS - 1.0)
    lo0 = jnp.floor(v0)
    lo1 = jnp.floor(v1)
    f0 = v0 - lo0
    f1 = v1 - lo1
    li0 = jnp.clip(lo0.astype(jnp.int32), 0, KS - 2)
    li1 = jnp.clip(lo1.astype(jnp.int32), 0, KS - 2)
    f0 = v0 - li0.astype(jnp.float32)
    f1 = v1 - li1.astype(jnp.float32)
    g0 = (src * KP + li1 * (KS - 1) + li0)
    gidx_ref[0:1, :] = g0
    gidx_ref[1:2, :] = g0 + (KS - 1)
    bw_ref[0:1, :] = (1.0 - f0) * (1.0 - f1)
    bw_ref[1:2, :] = f0 * (1.0 - f1)
    bw_ref[2:3, :] = (1.0 - f0) * f1
    bw_ref[3:4, :] = f0 * f1


def _mm_body(x_ref, w_ref, t_ref):
    y = jnp.dot(x_ref[...], w_ref[...], preferred_element_type=jnp.float32)
    for i1 in range(KS):
        for i0 in range(KS - 1):
            so = (i1 * KS + i0) * OUT_C
            do = (i1 * (KS - 1) + i0) * 2 * OUT_C
            t_ref[:, do:do + OUT_C] = y[:, so:so + OUT_C]
            t_ref[:, do + OUT_C:do + 2 * OUT_C] = y[:, so + OUT_C:so + 2 * OUT_C]


def _sc_body(y_ref, meta_ref, bw_ref, out_ref,
             acc, mbufA, mbufB, wbufA, wbufB, gbufA, gbufB, sbuf,
             dstA, dstB, semA, semB, semS, semMA, semMB):
    cid = lax.axis_index("c")
    sid = lax.axis_index("s")
    wid = cid * 16 + sid

    zero16 = jnp.zeros((LANES,), jnp.float32)

    def _zrow(i, _):
        for j in range(IN_C // LANES):
            sbuf[i, pl.ds(j * LANES, LANES)] = zero16
        return 0
    lax.fori_loop(0, CHUNK, _zrow, 0)

    rbase = sid * ROWS_PER_TILE
    nfull = ROWS_PER_TILE // CHUNK
    for i in range(nfull):
        pltpu.sync_copy(sbuf, acc.at[pl.ds(rbase + i * CHUNK, CHUNK)])
    rem = ROWS_PER_TILE - nfull * CHUNK
    if rem:
        pltpu.sync_copy(sbuf.at[pl.ds(0, rem)],
                        acc.at[pl.ds(rbase + nfull * CHUNK, rem)])

    plsc.subcore_barrier()

    def _start_gathers(mbuf, gbuf, sem):
        for p in range(2):
            for h in range(2):
                hsl = pl.ds(h * LANES, LANES)
                pltpu.async_copy(y_ref.at[mbuf.at[p, hsl]],
                                 gbuf.at[p, hsl], sem)

    def _drain_gathers(gbuf, sem):
        dummy = y_ref.at[pl.ds(0, LANES)]
        for p in range(2):
            for h in range(2):
                pltpu.make_async_copy(
                    dummy, gbuf.at[p, pl.ds(h * LANES, LANES)], sem).wait()

    def _drain_scatter(sbuf, sem):
        dummy = out_ref.at[0, pl.ds(0, CHUNK)]
        pltpu.make_async_copy(dummy, sbuf, sem).wait()

    def _compute(mbuf, wbuf, gbuf, dstv):
        def _egroup(g, _):
            gsl = pl.ds(g * LANES, LANES)
            wv = [wbuf[s, gsl] for s in range(S)]
            dstv[gsl] = mbuf[2, gsl]
            for i in range(LANES):
                e = g * LANES + i
                w0, w1, w2, w3 = (wv[s][i] for s in range(S))
                for j in range(IN_C // LANES):
                    sl = pl.ds(j * LANES, LANES)
                    sh = pl.ds(OUT_C + j * LANES, LANES)
                    v = (w0 * gbuf[0, e, sl] + w1 * gbuf[0, e, sh]
                         + w2 * gbuf[1, e, sl] + w3 * gbuf[1, e, sh])
                    sbuf[e, sl] = v
            return 0
        lax.fori_loop(0, CHUNK // LANES, _egroup, 0)

    def _drain_meta(mbuf, wbuf, sem):
        pltpu.make_async_copy(meta_ref.at[wid, 0], mbuf, sem).wait()
        pltpu.make_async_copy(bw_ref.at[wid, 0], wbuf, sem).wait()

    pltpu.sync_copy(meta_ref.at[wid, 0], mbufA)
    pltpu.sync_copy(bw_ref.at[wid, 0], wbufA)
    pltpu.async_copy(meta_ref.at[wid, 1], mbufB, semMB)
    pltpu.async_copy(bw_ref.at[wid, 1], wbufB, semMB)
    _start_gathers(mbufA, gbufA, semA)

    def _pair(p, _):
        a = 2 * p
        b = 2 * p + 1

        @pl.when(p > 0)
        def _():
            _drain_scatter(sbuf, semS)
        _drain_meta(mbufB, wbufB, semMB)
        _start_gathers(mbufB, gbufB, semB)

        _drain_gathers(gbufA, semA)
        _compute(mbufA, wbufA, gbufA, dstA)
        pltpu.async_copy(sbuf, acc.at[dstA], semS, add=True)

        @pl.when(a + 2 < NCH)
        def _():
            pltpu.async_copy(meta_ref.at[wid, a + 2], mbufA, semMA)
            pltpu.async_copy(bw_ref.at[wid, a + 2], wbufA, semMA)

        _drain_gathers(gbufB, semB)
        _drain_scatter(sbuf, semS)

        @pl.when(a + 2 < NCH)
        def _():
            _drain_meta(mbufA, wbufA, semMA)
            _start_gathers(mbufA, gbufA, semA)

        _compute(mbufB, wbufB, gbufB, dstB)
        pltpu.async_copy(sbuf, acc.at[dstB], semS, add=True)

        @pl.when(b + 2 < NCH)
        def _():
            pltpu.async_copy(meta_ref.at[wid, b + 2], mbufB, semMB)
            pltpu.async_copy(bw_ref.at[wid, b + 2], wbufB, semMB)
        return 0
    lax.fori_loop(0, NCH // 2, _pair, 0)

    _drain_scatter(sbuf, semS)

    plsc.subcore_barrier()
    pltpu.sync_copy(acc.at[pl.ds(rbase, ROWS_PER_TILE)],
                    out_ref.at[cid, pl.ds(rbase, ROWS_PER_TILE)])


DEG_BE = 2000


def _deg_body(dst_ref, deg_ref):
    @pl.when(pl.program_id(0) == 0)
    def _():
        deg_ref[...] = jnp.zeros_like(deg_ref)
    d2 = dst_ref[...]
    hi = (lax.shift_right_logical(d2, 7)
          == lax.broadcasted_iota(jnp.int32, (DEG_BE, NPAD // 128), 1)
          ).astype(jnp.float32)
    lo = ((d2 & 127)
          == lax.broadcasted_iota(jnp.int32, (DEG_BE, 128), 1)
          ).astype(jnp.float32)
    deg_ref[...] += lax.dot_general(hi, lo, (((0,), (0,)), ((), ())),
                                    preferred_element_type=jnp.float32)


def _final_body(x_ref, p0_ref, p1_ref, d_ref, wr_ref, b_ref,
                g_ref, be_ref, out_ref):
    msg = p0_ref[...] + p1_ref[...]
    deg = d_ref[...]
    msg = msg / jnp.maximum(deg, 1.0)
    out = msg + jnp.dot(x_ref[...], wr_ref[...],
                        preferred_element_type=jnp.float32) + b_ref[...]
    out = jnp.where(out > 0.0, out, jnp.exp(out) - 1.0)
    mean = jnp.mean(out, axis=0, keepdims=True)
    var = jnp.mean((out - mean) ** 2, axis=0, keepdims=True)
    out_ref[...] = (g_ref[...] * (out - mean) / jnp.sqrt(var + 1e-5)
                    + be_ref[...])


def kernel(x, edge_index, edge_attr, W, W_root, bias, gamma, beta):
    src = edge_index[0].reshape(1, E)
    dst = edge_index[1]
    attr_t = edge_attr.T

    gidx, bw = pl.pallas_call(
        _basis_body,
        out_shape=[jax.ShapeDtypeStruct((2, E), jnp.int32),
                   jax.ShapeDtypeStruct((S, E), jnp.float32)],
    )(attr_t, src)

    pad = ((0, 0), (0, 0), (0, EPW_PAD - EPW))
    gidx_p = jnp.pad(gidx.reshape(2, NW, EPW), pad)
    dst_p = jnp.pad(dst.reshape(1, NW, EPW), pad,
                    constant_values=NPAD - 1)
    meta = jnp.concatenate([gidx_p, dst_p], axis=0)
    meta_b = meta.reshape(3, NW, NCH, CHUNK).transpose(1, 2, 0, 3)
    bw_p = jnp.pad(bw.reshape(S, NW, EPW), pad)
    bw_b = bw_p.reshape(S, NW, NCH, CHUNK).transpose(1, 2, 0, 3)

    wf = W.transpose(1, 0, 2).reshape(IN_C, K * OUT_C)
    BN = 1000
    y = pl.pallas_call(
        _mm_body,
        grid=(N // BN,),
        in_specs=[pl.BlockSpec((BN, IN_C), lambda i: (i, 0)),
                  pl.BlockSpec((IN_C, K * OUT_C), lambda i: (0, 0))],
        out_specs=pl.BlockSpec((BN, KP * 2 * OUT_C), lambda i: (i, 0)),
        out_shape=jax.ShapeDtypeStruct((N, KP * 2 * OUT_C), jnp.float32),
    )(x, wf)
    y2 = y.reshape(N * KP, 2 * OUT_C)

    partials = pl.kernel(
        _sc_body,
        out_type=jax.ShapeDtypeStruct((2, NPAD, OUT_C), jnp.float32),
        mesh=plsc.VectorSubcoreMesh(core_axis_name="c",
                                    subcore_axis_name="s"),
        scratch_types=[
            pltpu.VMEM_SHARED((NPAD, OUT_C), jnp.float32),
            pltpu.VMEM((3, CHUNK), jnp.int32),
            pltpu.VMEM((3, CHUNK), jnp.int32),
            pltpu.VMEM((S, CHUNK), jnp.float32),
            pltpu.VMEM((S, CHUNK), jnp.float32),
            pltpu.VMEM((2, CHUNK, 2 * OUT_C), jnp.float32),
            pltpu.VMEM((2, CHUNK, 2 * OUT_C), jnp.float32),
            pltpu.VMEM((CHUNK, OUT_C), jnp.float32),
            pltpu.VMEM((CHUNK,), jnp.int32),
            pltpu.VMEM((CHUNK,), jnp.int32),
            pltpu.SemaphoreType.DMA,
            pltpu.SemaphoreType.DMA,
            pltpu.SemaphoreType.DMA,
            pltpu.SemaphoreType.DMA,
            pltpu.SemaphoreType.DMA,
        ],
    )(y2, meta_b, bw_b)

    deg = pl.pallas_call(
        _deg_body,
        grid=(E // DEG_BE,),
        in_specs=[pl.BlockSpec((DEG_BE, 1), lambda i: (i, 0))],
        out_specs=pl.BlockSpec((NPAD // 128, 128), lambda i: (0, 0)),
        out_shape=jax.ShapeDtypeStruct((NPAD // 128, 128), jnp.float32),
    )(dst.reshape(E, 1))

    p0 = partials[0, :N]
    p1 = partials[1, :N]
    degc = deg.reshape(NPAD, 1)[:N]

    out = pl.pallas_call(
        _final_body,
        out_shape=jax.ShapeDtypeStruct((N, OUT_C), jnp.float32),
    )(x, p0, p1, degc, W_root, bias.reshape(1, OUT_C),
      gamma.reshape(1, OUT_C), beta.reshape(1, OUT_C))
    return out


# speedup vs baseline: 1.5871x; 1.5871x over previous
import jax
import jax.numpy as jnp
from jax import lax
from jax.experimental import pallas as pl
from jax.experimental.pallas import tpu as pltpu
from jax.experimental.pallas import tpu_sc as plsc

N = 10000
E = 320000
IN_C = 128
OUT_C = 128
DIM = 2
KS = 5
K = KS ** DIM
S = 2 ** DIM

NW = 32
EPW = E // NW
CHUNK = 32
NCH = 314
EPW_PAD = NCH * CHUNK
NPAD = 10240
ROWS_PER_TILE = NPAD // 16
LANES = 16


def _basis_body(attr_ref, src_ref, gidx_ref, bw_ref):
    a0 = attr_ref[0:1, :]
    a1 = attr_ref[1:2, :]
    src = src_ref[0:1, :]
    v0 = a0 * (KS - 1.0)
    v1 = a1 * (KS - 1.0)
    lo0 = jnp.floor(v0)
    lo1 = jnp.floor(v1)
    f0 = v0 - lo0
    f1 = v1 - lo1
    li0 = lo0.astype(jnp.int32)
    li1 = lo1.astype(jnp.int32)
    for combo in range(S):
        b0 = combo & 1
        b1 = (combo >> 1) & 1
        i0 = jnp.clip(li0 + b0, 0, KS - 1)
        i1 = jnp.clip(li1 + b1, 0, KS - 1)
        w = (f0 if b0 else 1.0 - f0) * (f1 if b1 else 1.0 - f1)
        gidx_ref[combo:combo + 1, :] = src * K + i0 + i1 * KS
        bw_ref[combo:combo + 1, :] = w


def _mm_body(x_ref, w_ref, y_ref):
    y_ref[...] = jnp.dot(x_ref[...], w_ref[...],
                         preferred_element_type=jnp.float32)


def _sc_body(y_ref, meta_ref, bw_ref, out_ref,
             acc, mbufA, mbufB, wbufA, wbufB, gbufA, gbufB, sbuf,
             dstA, dstB, semA, semB, semS, semMA, semMB):
    cid = lax.axis_index("c")
    sid = lax.axis_index("s")
    wid = cid * 16 + sid

    zero16 = jnp.zeros((LANES,), jnp.float32)

    def _zrow(i, _):
        for j in range(IN_C // LANES):
            sbuf[i, pl.ds(j * LANES, LANES)] = zero16
        return 0
    lax.fori_loop(0, CHUNK, _zrow, 0)

    rbase = sid * ROWS_PER_TILE
    nfull = ROWS_PER_TILE // CHUNK
    for i in range(nfull):
        pltpu.sync_copy(sbuf, acc.at[pl.ds(rbase + i * CHUNK, CHUNK)])
    rem = ROWS_PER_TILE - nfull * CHUNK
    if rem:
        pltpu.sync_copy(sbuf.at[pl.ds(0, rem)],
                        acc.at[pl.ds(rbase + nfull * CHUNK, rem)])

    plsc.subcore_barrier()

    def _start_gathers(mbuf, gbuf, sem):
        for s in range(S):
            pltpu.async_copy(y_ref.at[mbuf.at[s]], gbuf.at[s], sem)

    def _drain_gathers(gbuf, sem):
        dummy = y_ref.at[pl.ds(0, CHUNK)]
        for s in range(S):
            pltpu.make_async_copy(dummy, gbuf.at[s], sem).wait()

    def _drain_scatter(sbuf, sem):
        dummy = out_ref.at[0, pl.ds(0, CHUNK)]
        pltpu.make_async_copy(dummy, sbuf, sem).wait()

    def _compute(mbuf, wbuf, gbuf, dstv):
        def _egroup(g, _):
            gsl = pl.ds(g * LANES, LANES)
            wv = [wbuf[s, gsl] for s in range(S)]
            dstv[gsl] = mbuf[S, gsl]
            for i in range(LANES):
                e = g * LANES + i
                w0, w1, w2, w3 = (wv[s][i] for s in range(S))
                for j in range(IN_C // LANES):
                    sl = pl.ds(j * LANES, LANES)
                    v = (w0 * gbuf[0, e, sl] + w1 * gbuf[1, e, sl]
                         + w2 * gbuf[2, e, sl] + w3 * gbuf[3, e, sl])
                    sbuf[e, sl] = v
            return 0
        lax.fori_loop(0, CHUNK // LANES, _egroup, 0)

    def _drain_meta(mbuf, wbuf, sem):
        pltpu.make_async_copy(meta_ref.at[wid, 0], mbuf, sem).wait()
        pltpu.make_async_copy(bw_ref.at[wid, 0], wbuf, sem).wait()

    pltpu.sync_copy(meta_ref.at[wid, 0], mbufA)
    pltpu.sync_copy(bw_ref.at[wid, 0], wbufA)
    pltpu.async_copy(meta_ref.at[wid, 1], mbufB, semMB)
    pltpu.async_copy(bw_ref.at[wid, 1], wbufB, semMB)
    _start_gathers(mbufA, gbufA, semA)

    def _pair(p, _):
        a = 2 * p
        b = 2 * p + 1

        @pl.when(p > 0)
        def _():
            _drain_scatter(sbuf, semS)
        _drain_meta(mbufB, wbufB, semMB)
        _start_gathers(mbufB, gbufB, semB)

        _drain_gathers(gbufA, semA)
        _compute(mbufA, wbufA, gbufA, dstA)
        pltpu.async_copy(sbuf, acc.at[dstA], semS, add=True)

        @pl.when(a + 2 < NCH)
        def _():
            pltpu.async_copy(meta_ref.at[wid, a + 2], mbufA, semMA)
            pltpu.async_copy(bw_ref.at[wid, a + 2], wbufA, semMA)

        _drain_gathers(gbufB, semB)
        _drain_scatter(sbuf, semS)

        @pl.when(a + 2 < NCH)
        def _():
            _drain_meta(mbufA, wbufA, semMA)
            _start_gathers(mbufA, gbufA, semA)

        _compute(mbufB, wbufB, gbufB, dstB)
        pltpu.async_copy(sbuf, acc.at[dstB], semS, add=True)

        @pl.when(b + 2 < NCH)
        def _():
            pltpu.async_copy(meta_ref.at[wid, b + 2], mbufB, semMB)
            pltpu.async_copy(bw_ref.at[wid, b + 2], wbufB, semMB)
        return 0
    lax.fori_loop(0, NCH // 2, _pair, 0)

    _drain_scatter(sbuf, semS)

    plsc.subcore_barrier()
    pltpu.sync_copy(acc.at[pl.ds(rbase, ROWS_PER_TILE)],
                    out_ref.at[cid, pl.ds(rbase, ROWS_PER_TILE)])


DEG_BE = 2000


def _deg_body(dst_ref, deg_ref):
    @pl.when(pl.program_id(0) == 0)
    def _():
        deg_ref[...] = jnp.zeros_like(deg_ref)
    d2 = dst_ref[...]
    hi = (lax.shift_right_logical(d2, 7)
          == lax.broadcasted_iota(jnp.int32, (DEG_BE, NPAD // 128), 1)
          ).astype(jnp.float32)
    lo = ((d2 & 127)
          == lax.broadcasted_iota(jnp.int32, (DEG_BE, 128), 1)
          ).astype(jnp.float32)
    deg_ref[...] += lax.dot_general(hi, lo, (((0,), (0,)), ((), ())),
                                    preferred_element_type=jnp.float32)


def _final_body(x_ref, p0_ref, p1_ref, d_ref, wr_ref, b_ref,
                g_ref, be_ref, out_ref):
    msg = p0_ref[...] + p1_ref[...]
    deg = d_ref[...]
    msg = msg / jnp.maximum(deg, 1.0)
    out = msg + jnp.dot(x_ref[...], wr_ref[...],
                        preferred_element_type=jnp.float32) + b_ref[...]
    out = jnp.where(out > 0.0, out, jnp.exp(out) - 1.0)
    mean = jnp.mean(out, axis=0, keepdims=True)
    var = jnp.mean((out - mean) ** 2, axis=0, keepdims=True)
    out_ref[...] = (g_ref[...] * (out - mean) / jnp.sqrt(var + 1e-5)
                    + be_ref[...])


def kernel(x, edge_index, edge_attr, W, W_root, bias, gamma, beta):
    src = edge_index[0].reshape(1, E)
    dst = edge_index[1]
    attr_t = edge_attr.T

    gidx, bw = pl.pallas_call(
        _basis_body,
        out_shape=[jax.ShapeDtypeStruct((S, E), jnp.int32),
                   jax.ShapeDtypeStruct((S, E), jnp.float32)],
    )(attr_t, src)

    pad = ((0, 0), (0, 0), (0, EPW_PAD - EPW))
    gidx_p = jnp.pad(gidx.reshape(S, NW, EPW), pad)
    dst_p = jnp.pad(dst.reshape(1, NW, EPW), pad,
                    constant_values=NPAD - 1)
    meta = jnp.concatenate([gidx_p, dst_p], axis=0)
    meta_b = meta.reshape(S + 1, NW, NCH, CHUNK).transpose(1, 2, 0, 3)
    bw_p = jnp.pad(bw.reshape(S, NW, EPW), pad)
    bw_b = bw_p.reshape(S, NW, NCH, CHUNK).transpose(1, 2, 0, 3)

    wf = W.transpose(1, 0, 2).reshape(IN_C, K * OUT_C)
    BN = 1000
    y = pl.pallas_call(
        _mm_body,
        grid=(N // BN,),
        in_specs=[pl.BlockSpec((BN, IN_C), lambda i: (i, 0)),
                  pl.BlockSpec((IN_C, K * OUT_C), lambda i: (0, 0))],
        out_specs=pl.BlockSpec((BN, K * OUT_C), lambda i: (i, 0)),
        out_shape=jax.ShapeDtypeStruct((N, K * OUT_C), jnp.float32),
    )(x, wf)
    y2 = y.reshape(N * K, OUT_C)

    partials = pl.kernel(
        _sc_body,
        out_type=jax.ShapeDtypeStruct((2, NPAD, OUT_C), jnp.float32),
        mesh=plsc.VectorSubcoreMesh(core_axis_name="c",
                                    subcore_axis_name="s"),
        scratch_types=[
            pltpu.VMEM_SHARED((NPAD, OUT_C), jnp.float32),
            pltpu.VMEM((S + 1, CHUNK), jnp.int32),
            pltpu.VMEM((S + 1, CHUNK), jnp.int32),
            pltpu.VMEM((S, CHUNK), jnp.float32),
            pltpu.VMEM((S, CHUNK), jnp.float32),
            pltpu.VMEM((S, CHUNK, IN_C), jnp.float32),
            pltpu.VMEM((S, CHUNK, IN_C), jnp.float32),
            pltpu.VMEM((CHUNK, OUT_C), jnp.float32),
            pltpu.VMEM((CHUNK,), jnp.int32),
            pltpu.VMEM((CHUNK,), jnp.int32),
            pltpu.SemaphoreType.DMA,
            pltpu.SemaphoreType.DMA,
            pltpu.SemaphoreType.DMA,
            pltpu.SemaphoreType.DMA,
            pltpu.SemaphoreType.DMA,
        ],
    )(y2, meta_b, bw_b)

    deg = pl.pallas_call(
        _deg_body,
        grid=(E // DEG_BE,),
        in_specs=[pl.BlockSpec((DEG_BE, 1), lambda i: (i, 0))],
        out_specs=pl.BlockSpec((NPAD // 128, 128), lambda i: (0, 0)),
        out_shape=jax.ShapeDtypeStruct((NPAD // 128, 128), jnp.float32),
    )(dst.reshape(E, 1))

    p0 = partials[0, :N]
    p1 = partials[1, :N]
    degc = deg.reshape(NPAD, 1)[:N]

    out = pl.pallas_call(
        _final_body,
        out_shape=jax.ShapeDtypeStruct((N, OUT_C), jnp.float32),
    )(x, p0, p1, degc, W_root, bias.reshape(1, OUT_C),
      gamma.reshape(1, OUT_C), beta.reshape(1, OUT_C))
    return out
